# baseline (speedup 1.0000x reference)
import functools

import jax
import jax.numpy as jnp
from jax import lax
from jax.experimental import pallas as pl
from jax.experimental.pallas import tpu as pltpu
from jax.experimental.pallas import tpu_sc as plsc

_VOCAB = 100002
_HIDDEN = 32
_SAMPLES = 20
_BATCH = 16384
_NC, _NS = 2, 16
_NW = _NC * _NS
_IPW = _BATCH // _NW
_MI = 128
_NMACRO = _IPW // _MI
_PACK = 128 // _HIDDEN

_sc_mesh = plsc.VectorSubcoreMesh(core_axis_name="c", subcore_axis_name="s")


@functools.partial(
    pl.kernel,
    mesh=_sc_mesh,
    out_type=jax.ShapeDtypeStruct((_SAMPLES, _BATCH, _HIDDEN), jnp.float32),
    scratch_types=[
        pltpu.VMEM((_SAMPLES, _MI), jnp.int32),
        pltpu.VMEM((_SAMPLES, _MI, _HIDDEN), jnp.float32),
        pltpu.SemaphoreType.DMA,
    ],
    compiler_params=pltpu.CompilerParams(
        use_tc_tiling_on_sc=False, needs_layout_passes=False
    ),
)
def _gather_sc(idx_hbm, table_hbm, out_hbm, idx_v, rows_v, sem):
    wid = lax.axis_index("s") * _NC + lax.axis_index("c")
    i0 = wid * _IPW

    def body(m, carry):
        r0 = i0 + m * _MI
        pltpu.sync_copy(idx_hbm.at[:, pl.ds(r0, _MI)], idx_v)
        copies = []
        for s in range(_SAMPLES):
            copies.append(
                pltpu.async_copy(
                    table_hbm.at[idx_v.at[s]], rows_v.at[s], sem
                )
            )
        for cp in copies:
            cp.wait()
        pltpu.sync_copy(rows_v, out_hbm.at[:, pl.ds(r0, _MI)])
        return carry

    lax.fori_loop(0, _NMACRO, body, 0)


def _tr_body(x_ref, o_ref):
    o_ref[...] = x_ref[...].T


_TRB = 2048


def _transpose_tc(all_values):
    return pl.pallas_call(
        _tr_body,
        grid=(_BATCH // _TRB,),
        in_specs=[pl.BlockSpec((_TRB, _SAMPLES), lambda i: (i, 0))],
        out_specs=pl.BlockSpec((_SAMPLES, _TRB), lambda i: (0, i)),
        out_shape=jax.ShapeDtypeStruct((_SAMPLES, _BATCH), jnp.int32),
    )(all_values)


def _mm_body(x_ref, bd_ref, b_ref, o_ref):
    acc = b_ref[...].astype(jnp.float32)
    for s in range(_SAMPLES):
        acc = acc + lax.dot_general(
            x_ref[s], bd_ref[s],
            (((1,), (0,)), ((), ())),
            preferred_element_type=jnp.float32,
        )
    o_ref[...] = acc


_BM4 = 512


def _project_tc(x3, BD, b128):
    return pl.pallas_call(
        _mm_body,
        grid=(_BATCH // _PACK // _BM4,),
        in_specs=[
            pl.BlockSpec((_SAMPLES, _BM4, 128), lambda i: (0, i, 0)),
            pl.BlockSpec((_SAMPLES, 128, 128), lambda i: (0, 0, 0)),
            pl.BlockSpec((1, 128), lambda i: (0, 0)),
        ],
        out_specs=pl.BlockSpec((_BM4, 128), lambda i: (i, 0)),
        out_shape=jax.ShapeDtypeStruct((_BATCH // _PACK, 128), jnp.float32),
    )(x3, BD, b128)


def kernel(all_values, table, W, b):
    idx_t = _transpose_tc(all_values)
    emb = _gather_sc(idx_t, table)
    x3 = emb.reshape(_SAMPLES, _BATCH // _PACK, 128)
    WsT = W.reshape(_HIDDEN, _SAMPLES, _HIDDEN).transpose(1, 2, 0)
    eye4 = jnp.eye(_PACK, dtype=W.dtype)
    BD = jnp.einsum("de,sfh->sdfeh", eye4, WsT).reshape(_SAMPLES, 128, 128)
    b128 = jnp.tile(b, _PACK).reshape(1, 128)
    out = _project_tc(x3, BD, b128)
    return out.reshape(_BATCH, _HIDDEN)


# speedup vs baseline: 1.0343x; 1.0343x over previous
import functools

import jax
import jax.numpy as jnp
from jax import lax
from jax.experimental import pallas as pl
from jax.experimental.pallas import tpu as pltpu
from jax.experimental.pallas import tpu_sc as plsc

_VOCAB = 100002
_HIDDEN = 32
_SAMPLES = 20
_BATCH = 16384
_NC, _NS = 2, 16
_NW = _NC * _NS
_IPW = _BATCH // _NW
_MI = 128
_NMACRO = _IPW // _MI
_PACK = 128 // _HIDDEN

_sc_mesh = plsc.VectorSubcoreMesh(core_axis_name="c", subcore_axis_name="s")


@functools.partial(
    pl.kernel,
    mesh=_sc_mesh,
    out_type=jax.ShapeDtypeStruct((_SAMPLES, _BATCH, _HIDDEN), jnp.float32),
    scratch_types=[
        pltpu.VMEM((_SAMPLES, _MI), jnp.int32),
        pltpu.VMEM((_SAMPLES, _MI, _HIDDEN), jnp.float32),
        pltpu.SemaphoreType.DMA,
    ],
    compiler_params=pltpu.CompilerParams(
        use_tc_tiling_on_sc=False, needs_layout_passes=False
    ),
)
def _gather_sc(idx_hbm, table_hbm, out_hbm, idx_v, rows_v, sem):
    wid = lax.axis_index("s") * _NC + lax.axis_index("c")
    i0 = wid * _IPW

    def body(m, carry):
        r0 = i0 + m * _MI
        pltpu.sync_copy(idx_hbm.at[:, pl.ds(r0, _MI)], idx_v)
        copies = []
        for s in range(_SAMPLES):
            copies.append(
                pltpu.async_copy(
                    table_hbm.at[idx_v.at[s]], rows_v.at[s], sem
                )
            )
        for cp in copies:
            cp.wait()
        pltpu.sync_copy(rows_v, out_hbm.at[:, pl.ds(r0, _MI)])
        return carry

    lax.fori_loop(0, _NMACRO, body, 0)


_VPAD = 100032
_V128 = _VPAD * _HIDDEN // 128
_CB = 4168


def _cv_body(x_ref, o_ref):
    for a in range(_PACK):
        o_ref[:, pl.ds(_HIDDEN * a, _HIDDEN)] = x_ref[
            pl.Slice(a, _CB, _PACK), :
        ]


def _compact_table_tc(table):
    return pl.pallas_call(
        _cv_body,
        grid=(_V128 // _CB,),
        in_specs=[pl.BlockSpec((_CB * _PACK, _HIDDEN), lambda i: (i, 0))],
        out_specs=pl.BlockSpec((_CB, 128), lambda i: (i, 0)),
        out_shape=jax.ShapeDtypeStruct((_V128, 128), jnp.float32),
    )(table)


def _mm_body(x_ref, bd_ref, b_ref, o_ref):
    acc = b_ref[...].astype(jnp.float32)
    for s in range(_SAMPLES):
        acc = acc + lax.dot_general(
            x_ref[s], bd_ref[s],
            (((1,), (0,)), ((), ())),
            preferred_element_type=jnp.float32,
        )
    o_ref[...] = acc


_BM4 = 512


def _project_tc(x3, BD, b128):
    return pl.pallas_call(
        _mm_body,
        grid=(_BATCH // _PACK // _BM4,),
        in_specs=[
            pl.BlockSpec((_SAMPLES, _BM4, 128), lambda i: (0, i, 0)),
            pl.BlockSpec((_SAMPLES, 128, 128), lambda i: (0, 0, 0)),
            pl.BlockSpec((1, 128), lambda i: (0, 0)),
        ],
        out_specs=pl.BlockSpec((_BM4, 128), lambda i: (i, 0)),
        out_shape=jax.ShapeDtypeStruct((_BATCH // _PACK, 128), jnp.float32),
    )(x3, BD, b128)


def kernel(all_values, table, W, b):
    idx_t = all_values.T
    tbl_lin = _compact_table_tc(table).reshape(_VPAD, _HIDDEN)
    emb = _gather_sc(idx_t, tbl_lin)
    x3 = emb.reshape(_SAMPLES, _BATCH // _PACK, 128)
    WsT = W.reshape(_HIDDEN, _SAMPLES, _HIDDEN).transpose(1, 2, 0)
    eye4 = jnp.eye(_PACK, dtype=W.dtype)
    BD = jnp.einsum("de,sfh->sdfeh", eye4, WsT).reshape(_SAMPLES, 128, 128)
    b128 = jnp.tile(b, _PACK).reshape(1, 128)
    out = _project_tc(x3, BD, b128)
    return out.reshape(_BATCH, _HIDDEN)


# speedup vs baseline: 1.0465x; 1.0117x over previous
import functools

import jax
import jax.numpy as jnp
from jax import lax
from jax.experimental import pallas as pl
from jax.experimental.pallas import tpu as pltpu
from jax.experimental.pallas import tpu_sc as plsc

_VOCAB = 100002
_HIDDEN = 32
_SAMPLES = 20
_BATCH = 16384
_NC, _NS = 2, 16
_NW = _NC * _NS
_IPW = _BATCH // _NW
_MI = 128
_NMACRO = _IPW // _MI
_PACK = 128 // _HIDDEN

_sc_mesh = plsc.VectorSubcoreMesh(core_axis_name="c", subcore_axis_name="s")


@functools.partial(
    pl.kernel,
    mesh=_sc_mesh,
    out_type=jax.ShapeDtypeStruct((_SAMPLES, _BATCH, _HIDDEN), jnp.float32),
    scratch_types=[
        pltpu.VMEM((_SAMPLES, _MI), jnp.int32),
        pltpu.VMEM((_SAMPLES, _MI, _HIDDEN), jnp.float32),
        pltpu.SemaphoreType.DMA,
    ],
    compiler_params=pltpu.CompilerParams(
        use_tc_tiling_on_sc=False, needs_layout_passes=False
    ),
)
def _gather_sc(idx_hbm, table_hbm, out_hbm, idx_v, rows_v, sem):
    wid = lax.axis_index("s") * _NC + lax.axis_index("c")
    i0 = wid * _IPW

    def body(m, carry):
        r0 = i0 + m * _MI
        pltpu.sync_copy(idx_hbm.at[:, pl.ds(r0, _MI)], idx_v)
        copies = []
        for s in range(_SAMPLES):
            copies.append(
                pltpu.async_copy(
                    table_hbm.at[idx_v.at[s]], rows_v.at[s], sem
                )
            )
        for cp in copies:
            cp.wait()
        pltpu.sync_copy(rows_v, out_hbm.at[:, pl.ds(r0, _MI)])
        return carry

    lax.fori_loop(0, _NMACRO, body, 0)


def _mm_body(x_ref, bd_ref, b_ref, o_ref):
    acc = b_ref[...].astype(jnp.float32)
    for s in range(_SAMPLES):
        acc = acc + lax.dot_general(
            x_ref[s], bd_ref[s],
            (((1,), (0,)), ((), ())),
            preferred_element_type=jnp.float32,
        )
    o_ref[...] = acc


_BM4 = 2048


def _project_tc(x3, BD, b128):
    return pl.pallas_call(
        _mm_body,
        grid=(_BATCH // _PACK // _BM4,),
        in_specs=[
            pl.BlockSpec((_SAMPLES, _BM4, 128), lambda i: (0, i, 0)),
            pl.BlockSpec((_SAMPLES, 128, 128), lambda i: (0, 0, 0)),
            pl.BlockSpec((1, 128), lambda i: (0, 0)),
        ],
        out_specs=pl.BlockSpec((_BM4, 128), lambda i: (i, 0)),
        out_shape=jax.ShapeDtypeStruct((_BATCH // _PACK, 128), jnp.float32),
    )(x3, BD, b128)


def kernel(all_values, table, W, b):
    idx_t = all_values.T
    emb = _gather_sc(idx_t, table)
    x3 = emb.reshape(_SAMPLES, _BATCH // _PACK, 128)
    WsT = W.reshape(_HIDDEN, _SAMPLES, _HIDDEN).transpose(1, 2, 0)
    eye4 = jnp.eye(_PACK, dtype=W.dtype)
    BD = jnp.einsum("de,sfh->sdfeh", eye4, WsT).reshape(_SAMPLES, 128, 128)
    b128 = jnp.tile(b, _PACK).reshape(1, 128)
    out = _project_tc(x3, BD, b128)
    return out.reshape(_BATCH, _HIDDEN)


# speedup vs baseline: 1.0536x; 1.0068x over previous
import functools

import jax
import jax.numpy as jnp
from jax import lax
from jax.experimental import pallas as pl
from jax.experimental.pallas import tpu as pltpu
from jax.experimental.pallas import tpu_sc as plsc

_VOCAB = 100002
_HIDDEN = 32
_SAMPLES = 20
_BATCH = 16384
_NC, _NS = 2, 16
_NW = _NC * _NS
_IPW = _BATCH // _NW
_MI = 128
_NMACRO = _IPW // _MI
_PACK = 128 // _HIDDEN

_sc_mesh = plsc.VectorSubcoreMesh(core_axis_name="c", subcore_axis_name="s")


@functools.partial(
    pl.kernel,
    mesh=_sc_mesh,
    out_type=jax.ShapeDtypeStruct((_SAMPLES, _BATCH, _HIDDEN), jnp.float32),
    scratch_types=[
        pltpu.VMEM((_SAMPLES, _MI), jnp.int32),
        pltpu.VMEM((_SAMPLES, _MI, _HIDDEN), jnp.float32),
        pltpu.SemaphoreType.DMA,
    ],
    compiler_params=pltpu.CompilerParams(
        use_tc_tiling_on_sc=False, needs_layout_passes=False
    ),
)
def _gather_sc(idx_hbm, table_hbm, out_hbm, idx_v, rows_v, sem):
    wid = lax.axis_index("s") * _NC + lax.axis_index("c")
    i0 = wid * _IPW

    def body(m, carry):
        r0 = i0 + m * _MI
        pltpu.sync_copy(idx_hbm.at[:, pl.ds(r0, _MI)], idx_v)
        copies = []
        for s in range(_SAMPLES):
            copies.append(
                pltpu.async_copy(
                    table_hbm.at[idx_v.at[s]], rows_v.at[s], sem
                )
            )
        for cp in copies:
            cp.wait()
        pltpu.sync_copy(rows_v, out_hbm.at[:, pl.ds(r0, _MI)])
        return carry

    lax.fori_loop(0, _NMACRO, body, 0)


def _mm_body(x_ref, bd_ref, b_ref, o_ref):
    acc = b_ref[...].astype(jnp.float32)
    for s in range(_SAMPLES):
        acc = acc + lax.dot_general(
            x_ref[s], bd_ref[s],
            (((1,), (0,)), ((), ())),
            preferred_element_type=jnp.float32,
        )
    o_ref[...] = acc


_BM4 = 512


def _project_tc(x3, BD, b128):
    return pl.pallas_call(
        _mm_body,
        grid=(_BATCH // _PACK // _BM4,),
        in_specs=[
            pl.BlockSpec((_SAMPLES, _BM4, 128), lambda i: (0, i, 0)),
            pl.BlockSpec((_SAMPLES, 128, 128), lambda i: (0, 0, 0)),
            pl.BlockSpec((1, 128), lambda i: (0, 0)),
        ],
        out_specs=pl.BlockSpec((_BM4, 128), lambda i: (i, 0)),
        out_shape=jax.ShapeDtypeStruct((_BATCH // _PACK, 128), jnp.float32),
    )(x3, BD, b128)


def kernel(all_values, table, W, b):
    idx_t = all_values.T
    emb = _gather_sc(idx_t, table)
    x3 = emb.reshape(_SAMPLES, _BATCH // _PACK, 128)
    WsT = W.reshape(_HIDDEN, _SAMPLES, _HIDDEN).transpose(1, 2, 0)
    eye4 = jnp.eye(_PACK, dtype=W.dtype)
    BD = jnp.einsum("de,sfh->sdfeh", eye4, WsT).reshape(_SAMPLES, 128, 128)
    b128 = jnp.tile(b, _PACK).reshape(1, 128)
    out = _project_tc(x3, BD, b128)
    return out.reshape(_BATCH, _HIDDEN)
